# baseline (speedup 1.0000x reference)
import functools

import jax
import jax.numpy as jnp
from jax import lax
from jax.experimental import pallas as pl
from jax.experimental.pallas import tpu as pltpu

VOCAB_CAT = 48
VOCAB_ATTR = 102
N_OBJ = 20
N_WORDS = 20


def _tc_body(cat_ref, attr_ref, out1_ref, out2_ref):
    bb = cat_ref.shape[0]
    rows = bb * N_OBJ

    iota1 = lax.broadcasted_iota(jnp.int32, (bb, N_OBJ, VOCAB_CAT), 2)
    out1_ref[...] = (cat_ref[...] == iota1).astype(jnp.float32)

    a = attr_ref[...].reshape(rows, N_WORDS)
    a = jnp.transpose(a, (1, 0))
    bit = jnp.left_shift(jnp.int32(1), a & 31)
    hi = jnp.right_shift(a, 5)

    def or_reduce0(x):
        s = x.shape[0]
        while s > 1:
            h = (s + 1) // 2
            x = x[:h] | x[s - h:s]
            s = h
        return x

    words = []
    for k in range(4):
        contrib = jnp.where(hi == k, bit, 0)
        words.append(or_reduce0(contrib))
    mask4 = jnp.concatenate(words, axis=0)
    mask4 = jnp.transpose(mask4, (1, 0))
    mask4 = mask4.reshape(bb, N_OBJ, 4)

    iota2 = lax.broadcasted_iota(jnp.int32, (bb, N_OBJ, VOCAB_ATTR), 2)
    sel = jnp.take_along_axis(mask4, jnp.right_shift(iota2, 5), axis=2)
    bits = jnp.right_shift(sel, iota2 & 31) & 1
    out2_ref[...] = bits.astype(jnp.float32)


@jax.jit
def kernel(category, attributes):
    B, N, _ = category.shape
    bb = 64
    grid = (B // bb,)
    return pl.pallas_call(
        _tc_body,
        grid=grid,
        in_specs=[
            pl.BlockSpec((bb, N, 1), lambda i: (i, 0, 0)),
            pl.BlockSpec((bb, N, N_WORDS), lambda i: (i, 0, 0)),
        ],
        out_specs=[
            pl.BlockSpec((bb, N, VOCAB_CAT), lambda i: (i, 0, 0)),
            pl.BlockSpec((bb, N, VOCAB_ATTR), lambda i: (i, 0, 0)),
        ],
        out_shape=[
            jax.ShapeDtypeStruct((B, N, VOCAB_CAT), jnp.float32),
            jax.ShapeDtypeStruct((B, N, VOCAB_ATTR), jnp.float32),
        ],
        compiler_params=pltpu.CompilerParams(
            dimension_semantics=("arbitrary",),
            vmem_limit_bytes=100 * 1024 * 1024),
    )(category, attributes)


# speedup vs baseline: 1.1041x; 1.1041x over previous
import functools

import jax
import jax.numpy as jnp
from jax import lax
from jax.experimental import pallas as pl
from jax.experimental.pallas import tpu as pltpu

VOCAB_CAT = 48
VOCAB_ATTR = 102
N_OBJ = 20
N_WORDS = 20


def _tc_body(cat_ref, attr_ref, out1_ref, out2_ref):
    bb = cat_ref.shape[0]
    rows = bb * N_OBJ

    iota1 = lax.broadcasted_iota(jnp.int32, (bb, N_OBJ, VOCAB_CAT), 2)
    out1_ref[...] = (cat_ref[...] == iota1).astype(jnp.float32)

    a = attr_ref[...].reshape(rows, N_WORDS)
    a = jnp.transpose(a, (1, 0))
    bit = jnp.left_shift(jnp.int32(1), a & 31)
    hi = jnp.right_shift(a, 5)

    def or_reduce0(x):
        s = x.shape[0]
        while s > 1:
            h = (s + 1) // 2
            x = x[:h] | x[s - h:s]
            s = h
        return x

    words = []
    for k in range(4):
        contrib = jnp.where(hi == k, bit, 0)
        words.append(or_reduce0(contrib))
    mask4 = jnp.concatenate(words, axis=0)
    mask4 = jnp.transpose(mask4, (1, 0))
    mask4 = mask4.reshape(bb, N_OBJ, 4)

    iota2 = lax.broadcasted_iota(jnp.int32, (bb, N_OBJ, VOCAB_ATTR), 2)
    sel = jnp.take_along_axis(mask4, jnp.right_shift(iota2, 5), axis=2)
    bits = jnp.right_shift(sel, iota2 & 31) & 1
    out2_ref[...] = bits.astype(jnp.float32)


@jax.jit
def kernel(category, attributes):
    B, N, _ = category.shape
    bb = 256
    grid = (B // bb,)
    return pl.pallas_call(
        _tc_body,
        grid=grid,
        in_specs=[
            pl.BlockSpec((bb, N, 1), lambda i: (i, 0, 0)),
            pl.BlockSpec((bb, N, N_WORDS), lambda i: (i, 0, 0)),
        ],
        out_specs=[
            pl.BlockSpec((bb, N, VOCAB_CAT), lambda i: (i, 0, 0)),
            pl.BlockSpec((bb, N, VOCAB_ATTR), lambda i: (i, 0, 0)),
        ],
        out_shape=[
            jax.ShapeDtypeStruct((B, N, VOCAB_CAT), jnp.float32),
            jax.ShapeDtypeStruct((B, N, VOCAB_ATTR), jnp.float32),
        ],
        compiler_params=pltpu.CompilerParams(
            dimension_semantics=("arbitrary",),
            vmem_limit_bytes=100 * 1024 * 1024),
    )(category, attributes)
